# baseline (speedup 1.0000x reference)
import functools

import jax
import jax.numpy as jnp
import numpy as np
from jax import lax
from jax.experimental import pallas as pl
from jax.experimental.pallas import tpu as pltpu
from jax.experimental.pallas import tpu_sc as plsc

BS = 4
AN = 10000
NEI = 16
GW = 128
KNUM = 64
N_FLAT = BS * AN

NC = 2
NS = 16
NW = NC * NS
WORKERS_PER_BATCH = NW // BS
ATOMS_PER_W = AN // WORKERS_PER_BATCH
GRP = 16
NGRP_PAIRS = 40
LAST_BASE = ATOMS_PER_W - GRP


def _sc_gather_angle(pos_f, fea_f, atom_mask, idx_w):
    mesh = plsc.VectorSubcoreMesh(core_axis_name="c", subcore_axis_name="s")
    SGW = GRP * GW

    @functools.partial(
        pl.kernel,
        out_type=jax.ShapeDtypeStruct((N_FLAT * GW,), jnp.float32),
        mesh=mesh,
        scratch_types=[
            pltpu.VMEM((AN * 3,), jnp.float32),
            pltpu.VMEM((AN * 5,), jnp.float32),
            pltpu.VMEM((AN,), jnp.float32),
            pltpu.VMEM((NEI * ATOMS_PER_W,), jnp.int32),
            pltpu.VMEM((2, SGW), jnp.float32),
            pltpu.SemaphoreType.DMA,
            pltpu.SemaphoreType.DMA,
        ],
        compiler_params=pltpu.CompilerParams(
            use_tc_tiling_on_sc=False, needs_layout_passes=False),
    )
    def sc_kernel(pos_hbm, fea_hbm, msk_hbm, idx_hbm, out_hbm,
                  pos_t, fea_t, msk_t, idx_b, stage, sw0, sw1):
        wid = lax.axis_index("s") * NC + lax.axis_index("c")
        bi = wid // WORKERS_PER_BATCH
        slot = wid % WORKERS_PER_BATCH
        row0 = bi * AN + slot * ATOMS_PER_W

        pltpu.sync_copy(pos_hbm.at[bi], pos_t)
        pltpu.sync_copy(fea_hbm.at[bi], fea_t)
        pltpu.sync_copy(msk_hbm.at[bi], msk_t)
        pltpu.sync_copy(
            idx_hbm.at[pl.ds(wid * NEI * ATOMS_PER_W, NEI * ATOMS_PER_W)],
            idx_b)

        zeros = jnp.zeros((GRP,), jnp.float32)
        for b in range(2):
            for k in range(SGW // GRP):
                stage.at[b][pl.ds(k * GRP, GRP)] = zeros

        iota = lax.iota(jnp.int32, GRP)
        rowoff = iota * GW
        sws = [sw0, sw1]

        def do_group(gi, b):
            base = jnp.minimum(gi * GRP, LAST_BASE)
            st = stage.at[b]
            own_i = slot * ATOMS_PER_W + base + iota
            own3 = own_i * 3
            ox = plsc.load_gather(pos_t, [own3])
            oy = plsc.load_gather(pos_t, [own3 + 1])
            oz = plsc.load_gather(pos_t, [own3 + 2])
            msk = plsc.load_gather(msk_t, [own_i])

            atom16 = (base + iota) * NEI
            d0x = d0y = d0z = inv0 = None
            for j in range(NEI):
                nbr = plsc.load_gather(idx_b, [atom16 + j])
                nbr3 = nbr * 3
                px = plsc.load_gather(pos_t, [nbr3])
                py = plsc.load_gather(pos_t, [nbr3 + 1])
                pz = plsc.load_gather(pos_t, [nbr3 + 2])
                dx, dy, dz = px - ox, py - oy, pz - oz
                n2 = jnp.maximum(dx * dx + dy * dy + dz * dz, 1e-24)
                i32 = plsc.bitcast(n2, jnp.int32)
                est = jnp.full((GRP,), 0x5F3759DF, jnp.int32) - (
                    jnp.right_shift(i32, 1))
                y = plsc.bitcast(est, jnp.float32)
                y = y * (1.5 - 0.5 * n2 * y * y)
                y = y * (1.5 - 0.5 * n2 * y * y)
                if j == 0:
                    d0x, d0y, d0z, inv0 = dx, dy, dz, y
                    theta = msk
                else:
                    numer = dx * d0x + dy * d0y + dz * d0z
                    theta = numer * y * inv0 * msk
                plsc.store_scatter(st, [rowoff + (80 + j)], theta)
                nbr5 = nbr * 5
                for c in range(5):
                    f = plsc.load_gather(fea_t, [nbr5 + c])
                    plsc.store_scatter(st, [rowoff + (j * 5 + c)], f)
            pltpu.async_copy(
                st, out_hbm.at[pl.ds((row0 + base) * GW, SGW)], sws[b])

        def pair(t, _):
            for b in range(2):
                @pl.when(t >= 1)
                def _wait():
                    pltpu.make_async_copy(
                        stage.at[b], out_hbm.at[pl.ds(row0 * GW, SGW)],
                        sws[b]).wait()
                do_group(2 * t + b, b)
            return _

        lax.fori_loop(0, NGRP_PAIRS, pair, None)
        for b in range(2):
            pltpu.make_async_copy(
                stage.at[b], out_hbm.at[pl.ds(row0 * GW, SGW)], sws[b]).wait()

    return sc_kernel(pos_f, fea_f, atom_mask, idx_w)


A_BLK = 1000


def _build_constants(angle_weight, scalar_weight, r1, r2):
    f32 = jnp.float32
    eye16 = jnp.eye(NEI, dtype=f32)

    msk16 = np.zeros((16, NEI), np.float32); msk16[8, :] = 1.0
    msk64 = np.zeros((16, KNUM), np.float32); msk64[8, :] = 1.0

    b1 = jnp.kron(eye16, r1.astype(f32))
    b2 = jnp.kron(eye16, r2.astype(f32))

    t4 = np.zeros((NEI, GW), np.float32)
    for j in range(NEI):
        t4[j, j * 5: j * 5 + 5] = 1.0
    w_nb = np.zeros((GW, KNUM), np.float32)
    w_nb = jnp.asarray(w_nb).at[0:5 * NEI].set(
        jnp.tile(scalar_weight[5:10].astype(f32), (NEI, 1)))

    sta = jnp.zeros((GW, KNUM), f32).at[80:96].set(angle_weight.astype(f32).T)

    w_own = jnp.zeros((16, KNUM), f32).at[3:8].set(scalar_weight[0:5].astype(f32))
    ones_g = jnp.ones((NEI, KNUM), f32)
    return dict(
        msk16=jnp.asarray(msk16), msk64=jnp.asarray(msk64),
        b1=b1, b2=b2, t4=jnp.asarray(t4),
        w_nb=w_nb, w_own=w_own, ones_g=ones_g, sta=sta,
    )


def _gate_body(ef_ref, own_ref, b1_ref, b2_ref, m16_ref, gate_ref):
    f32 = jnp.float32
    dot = functools.partial(jnp.dot, preferred_element_type=f32)
    ef = ef_ref[...]
    m16 = dot(own_ref[...], m16_ref[...])
    a = jnp.maximum(dot(ef, b1_ref[...]), 0.0)
    b = jnp.maximum(dot(a, b2_ref[...]), 0.0)
    gate_ref[...] = jax.nn.sigmoid(b * m16)


def _tc_gate(ef, own, c):
    grid = (N_FLAT // A_BLK,)
    full = lambda shape: pl.BlockSpec(shape, lambda i: (0,) * len(shape))
    return pl.pallas_call(
        _gate_body,
        grid=grid,
        in_specs=[
            pl.BlockSpec((A_BLK, 2 * NEI), lambda i: (i, 0)),
            pl.BlockSpec((A_BLK, 16), lambda i: (i, 0)),
            full(c["b1"].shape), full(c["b2"].shape), full(c["msk16"].shape),
        ],
        out_specs=pl.BlockSpec((A_BLK, NEI), lambda i: (i, 0)),
        out_shape=jax.ShapeDtypeStruct((N_FLAT, NEI), jnp.float32),
    )(ef, own, c["b1"], c["b2"], c["msk16"])


def _tc_body(row_ref, own_ref, gate_ref,
             m64_ref, t4_ref, wnb_ref, wown_ref, ones_ref, sta_ref,
             out_ref):
    f32 = jnp.float32
    dot = functools.partial(jnp.dot, preferred_element_type=f32)
    g = row_ref[...]
    own = own_ref[...]
    gate = gate_ref[...]
    m64 = dot(own, m64_ref[...])

    struct = dot(g, sta_ref[...])
    g128 = dot(gate, t4_ref[...])
    nb_elem = dot(g128 * g, wnb_ref[...])
    own_elem = dot(own, wown_ref[...])
    gsum = dot(gate, ones_ref[...])
    fea_elem = (gsum * own_elem + nb_elem) * m64

    x = fea_elem + struct
    out_ref[...] = jnp.where(x >= 0, x, 0.01 * x) * m64


def _tc_dense(rows, own, gate, c):
    grid = (N_FLAT // A_BLK,)
    full = lambda shape: pl.BlockSpec(shape, lambda i: (0,) * len(shape))
    consts = [c["msk64"], c["t4"], c["w_nb"], c["w_own"],
              c["ones_g"], c["sta"]]
    return pl.pallas_call(
        _tc_body,
        grid=grid,
        in_specs=[
            pl.BlockSpec((A_BLK, GW), lambda i: (i, 0)),
            pl.BlockSpec((A_BLK, 16), lambda i: (i, 0)),
            pl.BlockSpec((A_BLK, NEI), lambda i: (i, 0)),
        ] + [full(x.shape) for x in consts],
        out_specs=pl.BlockSpec((A_BLK, KNUM), lambda i: (i, 0)),
        out_shape=jax.ShapeDtypeStruct((N_FLAT, KNUM), jnp.float32),
    )(rows, own, gate, *consts)


def kernel(pos, atom_fea, edge_index, edge_fea, atom_mask,
           angle_weight, scalar_weight, radius_weight_1, radius_weight_2):
    own = jnp.concatenate(
        [pos, atom_fea, atom_mask[..., None],
         jnp.zeros((BS, AN, 7), jnp.float32)], axis=-1)
    idx_w = edge_index.reshape(BS * AN * NEI)

    c = _build_constants(angle_weight, scalar_weight,
                         radius_weight_1, radius_weight_2)
    rows = _sc_gather_angle(pos.reshape(BS, AN * 3),
                            atom_fea.reshape(BS, AN * 5),
                            atom_mask, idx_w).reshape(N_FLAT, GW)
    gate = _tc_gate(edge_fea.reshape(N_FLAT, 2 * NEI),
                    own.reshape(N_FLAT, 16), c)
    out = _tc_dense(rows, own.reshape(N_FLAT, 16), gate, c)
    return out.reshape(BS, AN, KNUM)
